# Initial kernel scaffold; baseline (speedup 1.0000x reference)
import jax
import jax.numpy as jnp
from jax.experimental import pallas as pl


def kernel(x, edge_index, agent_features, enc_W, enc_b, enc_g, enc_beta, virtual_node, gat_W0, att_src0, att_dst0, gat_b0, ln_g0, ln_b0, gat_W1, att_src1, att_dst1, gat_b1, ln_g1, ln_b1, gat_W2, att_src2, att_dst2, gat_b2, ln_g2, ln_b2, ag_W, ag_b, ag_g, ag_beta, v1_W, v1_b, v_g, v_beta, v2_W, v2_b, a1_W, a1_b, a_g, a_beta, a2_W, a2_b):
    raise NotImplementedError("write your pallas kernel here")


# speedup vs baseline: 1.4988x; 1.4988x over previous
import jax
import jax.numpy as jnp
from jax.experimental import pallas as pl

N = 50000
NP1 = N + 1
T = 1024
NT = 49
NPAD = NT * T
NEG = -1e30


def _ln_in(t, g, b):
    mu = jnp.mean(t, axis=-1, keepdims=True)
    var = jnp.mean((t - mu) ** 2, axis=-1, keepdims=True)
    return (t - mu) / jnp.sqrt(var + 1e-5) * g + b


def _enc_kernel(x_ref, ew_ref, eb_ref, eg_ref, ebt_ref, vn_ref, w0_ref,
                xv_ref, hsd_ref):
    i = pl.program_id(0)
    xb = x_ref[...]
    t = jnp.dot(xb, ew_ref[...], preferred_element_type=jnp.float32) + eb_ref[...]
    t = jnp.maximum(_ln_in(t, eg_ref[...], ebt_ref[...]), 0.0)
    gidx = i * T + jax.lax.broadcasted_iota(jnp.int32, (T, 1), 0)
    t = jnp.where(gidx == N, vn_ref[...], t)
    xv_ref[...] = t
    hsd_ref[...] = jnp.dot(t, w0_ref[...], preferred_element_type=jnp.float32)


def _fused_kernel(seg_ref, hp_ref, b_ref, g_ref, bt_ref, wn_ref,
                  hnew_ref, hsd_ref):
    t = _ln_in(seg_ref[...] + b_ref[...], g_ref[...], bt_ref[...])
    hnew = jnp.maximum(t + hp_ref[...], 0.0)
    hnew_ref[...] = hnew
    hsd_ref[...] = jnp.dot(hnew, wn_ref[...], preferred_element_type=jnp.float32)


def _flash_kernel(hsd_ref, rowv_ref, st_ref):
    p = pl.program_id(0)
    i = pl.program_id(1)

    @pl.when((p == 0) & (i == 0))
    def _init():
        st_ref[...] = jnp.zeros((8, 128), jnp.float32)
        st_ref[5:6, :] = jnp.full((1, 128), NEG, jnp.float32)

    blk = hsd_ref[...]
    gidx = i * T + jax.lax.broadcasted_iota(jnp.int32, (T, 1), 0)
    valid = gidx < NP1

    @pl.when(p == 0)
    def _maxpass():
        mrow = st_ref[5:6, :]
        lane = jax.lax.broadcasted_iota(jnp.int32, (1, 128), 1)
        for k in range(4):
            slk = blk[:, 64 + k:65 + k]
            dvk = rowv_ref[0, 68 + k]
            alk = slk + dvk
            alk = jnp.where(alk >= 0, alk, 0.2 * alk)
            alk = jnp.where(valid, alk, NEG)
            mk = jnp.max(alk)
            mrow_new = jnp.where(lane == k, jnp.maximum(mrow, mk), mrow)
            mrow = mrow_new
        st_ref[5:6, :] = mrow

    @pl.when(p == 1)
    def _sumpass():
        for k in range(4):
            slk = blk[:, 64 + k:65 + k]
            dvk = rowv_ref[0, 68 + k]
            alk = slk + dvk
            alk = jnp.where(alk >= 0, alk, 0.2 * alk)
            alk = jnp.where(valid, alk, NEG)
            mk = st_ref[5, k]
            ek = jnp.exp(alk - mk)
            ek = jnp.where(valid, ek, 0.0)
            dsum = jnp.sum(ek)
            lane = jax.lax.broadcasted_iota(jnp.int32, (1, 128), 1)
            drow = st_ref[4:5, :]
            st_ref[4:5, :] = jnp.where(lane == k, drow + dsum, drow)
            hk = blk[:, 16 * k:16 * (k + 1)]
            st_ref[k:k + 1, 0:16] = st_ref[k:k + 1, 0:16] + jnp.sum(
                ek * hk, axis=0, keepdims=True)


def _head_kernel(st_ref, jk_ref, h2v_ref, gb2_ref, lg2_ref, lb2_ref,
                 agf_ref, agw_ref, agb_ref, agg_ref, agbt_ref,
                 v1a_ref, v1b_ref, v1c_ref, v1bias_ref, vg_ref, vbt_ref,
                 v2w_ref, v2b_ref,
                 a1a_ref, a1b_ref, a1c_ref, a1bias_ref, ag2_ref, abt2_ref,
                 a2w_ref, a2b_ref, out_ref):
    numer = st_ref[0:4, 0:16]
    denom = st_ref[4:5, 0:4]
    outv = numer / (denom.reshape(4, 1) + 1e-16)
    outv = jnp.concatenate(
        [outv[0:1, :], outv[1:2, :], outv[2:3, :], outv[3:4, :]], axis=1)
    hn = _ln_in(outv + gb2_ref[...], lg2_ref[...], lb2_ref[...])
    h3v = jnp.maximum(hn + h2v_ref[...], 0.0)

    ag = jnp.dot(agf_ref[...], agw_ref[...],
                 preferred_element_type=jnp.float32) + agb_ref[...]
    ag = jnp.maximum(_ln_in(ag, agg_ref[...], agbt_ref[...]), 0.0)

    jk = jk_ref[...]

    def branch(wa, wb, wc, b1, g1, bt1, w2, b2):
        pre = (jnp.dot(jk, wa, preferred_element_type=jnp.float32)
               + jnp.dot(h3v, wb, preferred_element_type=jnp.float32)
               + jnp.dot(ag, wc, preferred_element_type=jnp.float32) + b1)
        pre = jnp.maximum(_ln_in(pre, g1, bt1), 0.0)
        return jnp.dot(pre, w2, preferred_element_type=jnp.float32) + b2

    vout = branch(v1a_ref[...], v1b_ref[...], v1c_ref[...], v1bias_ref[...],
                  vg_ref[...], vbt_ref[...], v2w_ref[...], v2b_ref[...])
    aout = branch(a1a_ref[...], a1b_ref[...], a1c_ref[...], a1bias_ref[...],
                  ag2_ref[...], abt2_ref[...], a2w_ref[...], a2b_ref[...])
    lane = jax.lax.broadcasted_iota(jnp.int32, (1, 128), 1)
    amean = jnp.sum(jnp.where(lane < 9, aout, 0.0)) / 9.0
    val = vout[0, 0]
    res = val + aout - amean
    out_ref[...] = jnp.broadcast_to(res, (8, 128))


def _headmat(a):
    eye = jnp.eye(4, dtype=a.dtype)
    return (a[:, :, None] * eye[:, None, :]).reshape(64, 4)


def _row(v):
    return v.reshape(1, -1)


def kernel(x, edge_index, agent_features, enc_W, enc_b, enc_g, enc_beta,
           virtual_node, gat_W0, att_src0, att_dst0, gat_b0, ln_g0, ln_b0,
           gat_W1, att_src1, att_dst1, gat_b1, ln_g1, ln_b1,
           gat_W2, att_src2, att_dst2, gat_b2, ln_g2, ln_b2,
           ag_W, ag_b, ag_g, ag_beta, v1_W, v1_b, v_g, v_beta, v2_W, v2_b,
           a1_W, a1_b, a_g, a_beta, a2_W, a2_b):
    f32 = jnp.float32
    x_pad = jnp.pad(x, ((0, NPAD - N), (0, 0)))
    def _wp(W, a_s, a_d):
        return jnp.concatenate(
            [W, W @ _headmat(a_s), W @ _headmat(a_d)], axis=1)

    W0p = _wp(gat_W0, att_src0, att_dst0)
    W1p = _wp(gat_W1, att_src1, att_dst1)
    W2p = _wp(gat_W2, att_src2, att_dst2)

    tile64 = pl.BlockSpec((T, 64), lambda i: (i, 0))
    tile72 = pl.BlockSpec((T, 72), lambda i: (i, 0))

    def full(a):
        return pl.BlockSpec(a.shape, lambda i: tuple(0 for _ in a.shape))

    xv_pad, hsd0 = pl.pallas_call(
        _enc_kernel,
        grid=(NT,),
        in_specs=[pl.BlockSpec((T, 8), lambda i: (i, 0)),
                  full(enc_W), full(_row(enc_b)), full(_row(enc_g)),
                  full(_row(enc_beta)), full(_row(virtual_node)), full(W0p)],
        out_specs=[tile64, tile72],
        out_shape=[jax.ShapeDtypeStruct((NPAD, 64), f32),
                   jax.ShapeDtypeStruct((NPAD, 72), f32)],
    )(x_pad, enc_W, _row(enc_b), _row(enc_g), _row(enc_beta),
      _row(virtual_node), W0p)

    idt = edge_index.dtype
    nodes = jnp.arange(N, dtype=idt)
    vfull = jnp.full((N,), N, dtype=idt)
    loop = jnp.arange(NP1, dtype=idt)
    s_full = jnp.concatenate([edge_index[0], vfull, nodes, loop])
    d_full = jnp.concatenate([edge_index[1], nodes, vfull, loop])

    def edge_pass(hsd):
        h = hsd[:NP1, :64]
        sl = hsd[:NP1, 64:68]
        dl = hsd[:NP1, 68:72]
        al = jax.nn.leaky_relu(sl[s_full] + dl[d_full], 0.2)
        m = jax.ops.segment_max(al, d_full, num_segments=NP1)
        e = jnp.exp(al - m[d_full])
        ssum = jax.ops.segment_sum(e, d_full, num_segments=NP1)
        alpha = e / (ssum[d_full] + 1e-16)
        msg = h.reshape(NP1, 4, 16)[s_full] * alpha[:, :, None]
        out = jax.ops.segment_sum(msg, d_full, num_segments=NP1)
        return jnp.pad(out.reshape(NP1, 64), ((0, NPAD - NP1), (0, 0)))

    def fused(seg, hprev, b, g, bt, Wn):
        return pl.pallas_call(
            _fused_kernel,
            grid=(NT,),
            in_specs=[tile64, tile64, full(_row(b)), full(_row(g)),
                      full(_row(bt)), full(Wn)],
            out_specs=[tile64, tile72],
            out_shape=[jax.ShapeDtypeStruct((NPAD, 64), f32),
                       jax.ShapeDtypeStruct((NPAD, 72), f32)],
        )(seg, hprev, _row(b), _row(g), _row(bt), Wn)

    seg0 = edge_pass(hsd0)
    h1_pad, hsd1 = fused(seg0, xv_pad, gat_b0, ln_g0, ln_b0, W1p)
    seg1 = edge_pass(hsd1)
    h2_pad, hsd2 = fused(seg1, h1_pad, gat_b1, ln_g1, ln_b1, W2p)

    rowv = jax.lax.dynamic_slice(hsd2, (N, 0), (1, 72))
    state = pl.pallas_call(
        _flash_kernel,
        grid=(2, NT),
        in_specs=[pl.BlockSpec((T, 72), lambda p, i: (i, 0)),
                  pl.BlockSpec((1, 72), lambda p, i: (0, 0))],
        out_specs=pl.BlockSpec((8, 128), lambda p, i: (0, 0)),
        out_shape=jax.ShapeDtypeStruct((8, 128), f32),
    )(hsd2, rowv)

    jk192 = jnp.concatenate(
        [xv_pad[N:N + 1], h1_pad[N:N + 1], h2_pad[N:N + 1]], axis=1)
    v2Wp = jnp.pad(v2_W, ((0, 0), (0, 127)))
    v2bp = jnp.pad(_row(v2_b), ((0, 0), (0, 127)))
    a2Wp = jnp.pad(a2_W, ((0, 0), (0, 119)))
    a2bp = jnp.pad(_row(a2_b), ((0, 0), (0, 119)))

    out = pl.pallas_call(
        _head_kernel,
        out_shape=jax.ShapeDtypeStruct((8, 128), f32),
    )(state, jk192, h2_pad[N:N + 1], _row(gat_b2), _row(ln_g2), _row(ln_b2),
      agent_features, ag_W, _row(ag_b), _row(ag_g), _row(ag_beta),
      v1_W[:192], v1_W[192:256], v1_W[256:], _row(v1_b), _row(v_g),
      _row(v_beta), v2Wp, v2bp,
      a1_W[:192], a1_W[192:256], a1_W[256:], _row(a1_b), _row(a_g),
      _row(a_beta), a2Wp, a2bp)
    return out[0:1, 0:9]


# speedup vs baseline: 10.0055x; 6.6758x over previous
import jax
import jax.numpy as jnp
from jax.experimental import pallas as pl

N = 50000
NP1 = N + 1
T = 1024
NT = 49
NPAD = NT * T
NEG = -1e30


def _ln_in(t, g, b):
    mu = jnp.mean(t, axis=-1, keepdims=True)
    var = jnp.mean((t - mu) ** 2, axis=-1, keepdims=True)
    return (t - mu) / jnp.sqrt(var + 1e-5) * g + b


def _enc_kernel(x_ref, ew_ref, eb_ref, eg_ref, ebt_ref, vn_ref, w0_ref,
                xv_ref, hsd_ref):
    i = pl.program_id(0)
    xb = x_ref[...]
    t = jnp.dot(xb, ew_ref[...], preferred_element_type=jnp.float32) + eb_ref[...]
    t = jnp.maximum(_ln_in(t, eg_ref[...], ebt_ref[...]), 0.0)
    gidx = i * T + jax.lax.broadcasted_iota(jnp.int32, (T, 1), 0)
    t = jnp.where(gidx == N, vn_ref[...], t)
    xv_ref[...] = t
    hsd_ref[...] = jnp.dot(t, w0_ref[...], preferred_element_type=jnp.float32)


def _fused_kernel(seg_ref, hp_ref, b_ref, g_ref, bt_ref, wn_ref,
                  hnew_ref, hsd_ref):
    blk = seg_ref[...]
    gat = jnp.concatenate(
        [blk[:, 16 * k:16 * (k + 1)] / (blk[:, 64 + k:65 + k] + 1e-16)
         for k in range(4)], axis=1)
    t = _ln_in(gat + b_ref[...], g_ref[...], bt_ref[...])
    hnew = jnp.maximum(t + hp_ref[...], 0.0)
    hnew_ref[...] = hnew
    hsd_ref[...] = jnp.dot(hnew, wn_ref[...], preferred_element_type=jnp.float32)


def _flash_kernel(hsd_ref, rowv_ref, st_ref):
    p = pl.program_id(0)
    i = pl.program_id(1)

    @pl.when((p == 0) & (i == 0))
    def _init():
        st_ref[...] = jnp.zeros((8, 128), jnp.float32)
        st_ref[5:6, :] = jnp.full((1, 128), NEG, jnp.float32)

    blk = hsd_ref[...]
    gidx = i * T + jax.lax.broadcasted_iota(jnp.int32, (T, 1), 0)
    valid = gidx < NP1

    @pl.when(p == 0)
    def _maxpass():
        mrow = st_ref[5:6, :]
        lane = jax.lax.broadcasted_iota(jnp.int32, (1, 128), 1)
        for k in range(4):
            slk = blk[:, 64 + k:65 + k]
            dvk = rowv_ref[0, 68 + k]
            alk = slk + dvk
            alk = jnp.where(alk >= 0, alk, 0.2 * alk)
            alk = jnp.where(valid, alk, NEG)
            mk = jnp.max(alk)
            mrow_new = jnp.where(lane == k, jnp.maximum(mrow, mk), mrow)
            mrow = mrow_new
        st_ref[5:6, :] = mrow

    @pl.when(p == 1)
    def _sumpass():
        for k in range(4):
            slk = blk[:, 64 + k:65 + k]
            dvk = rowv_ref[0, 68 + k]
            alk = slk + dvk
            alk = jnp.where(alk >= 0, alk, 0.2 * alk)
            alk = jnp.where(valid, alk, NEG)
            mk = st_ref[5, k]
            ek = jnp.exp(alk - mk)
            ek = jnp.where(valid, ek, 0.0)
            dsum = jnp.sum(ek)
            lane = jax.lax.broadcasted_iota(jnp.int32, (1, 128), 1)
            drow = st_ref[4:5, :]
            st_ref[4:5, :] = jnp.where(lane == k, drow + dsum, drow)
            hk = blk[:, 16 * k:16 * (k + 1)]
            st_ref[k:k + 1, 0:16] = st_ref[k:k + 1, 0:16] + jnp.sum(
                ek * hk, axis=0, keepdims=True)


def _head_kernel(st_ref, jk_ref, h2v_ref, gb2_ref, lg2_ref, lb2_ref,
                 agf_ref, agw_ref, agb_ref, agg_ref, agbt_ref,
                 v1a_ref, v1b_ref, v1c_ref, v1bias_ref, vg_ref, vbt_ref,
                 v2w_ref, v2b_ref,
                 a1a_ref, a1b_ref, a1c_ref, a1bias_ref, ag2_ref, abt2_ref,
                 a2w_ref, a2b_ref, out_ref):
    numer = st_ref[0:4, 0:16]
    denom = st_ref[4:5, 0:4]
    outv = numer / (denom.reshape(4, 1) + 1e-16)
    outv = jnp.concatenate(
        [outv[0:1, :], outv[1:2, :], outv[2:3, :], outv[3:4, :]], axis=1)
    hn = _ln_in(outv + gb2_ref[...], lg2_ref[...], lb2_ref[...])
    h3v = jnp.maximum(hn + h2v_ref[...], 0.0)

    ag = jnp.dot(agf_ref[...], agw_ref[...],
                 preferred_element_type=jnp.float32) + agb_ref[...]
    ag = jnp.maximum(_ln_in(ag, agg_ref[...], agbt_ref[...]), 0.0)

    jk = jk_ref[...]

    def branch(wa, wb, wc, b1, g1, bt1, w2, b2):
        pre = (jnp.dot(jk, wa, preferred_element_type=jnp.float32)
               + jnp.dot(h3v, wb, preferred_element_type=jnp.float32)
               + jnp.dot(ag, wc, preferred_element_type=jnp.float32) + b1)
        pre = jnp.maximum(_ln_in(pre, g1, bt1), 0.0)
        return jnp.dot(pre, w2, preferred_element_type=jnp.float32) + b2

    vout = branch(v1a_ref[...], v1b_ref[...], v1c_ref[...], v1bias_ref[...],
                  vg_ref[...], vbt_ref[...], v2w_ref[...], v2b_ref[...])
    aout = branch(a1a_ref[...], a1b_ref[...], a1c_ref[...], a1bias_ref[...],
                  ag2_ref[...], abt2_ref[...], a2w_ref[...], a2b_ref[...])
    lane = jax.lax.broadcasted_iota(jnp.int32, (1, 128), 1)
    amean = jnp.sum(jnp.where(lane < 9, aout, 0.0)) / 9.0
    val = vout[0, 0]
    res = val + aout - amean
    out_ref[...] = jnp.broadcast_to(res, (8, 128))


def _headmat(a):
    eye = jnp.eye(4, dtype=a.dtype)
    return (a[:, :, None] * eye[:, None, :]).reshape(64, 4)


def _row(v):
    return v.reshape(1, -1)


def kernel(x, edge_index, agent_features, enc_W, enc_b, enc_g, enc_beta,
           virtual_node, gat_W0, att_src0, att_dst0, gat_b0, ln_g0, ln_b0,
           gat_W1, att_src1, att_dst1, gat_b1, ln_g1, ln_b1,
           gat_W2, att_src2, att_dst2, gat_b2, ln_g2, ln_b2,
           ag_W, ag_b, ag_g, ag_beta, v1_W, v1_b, v_g, v_beta, v2_W, v2_b,
           a1_W, a1_b, a_g, a_beta, a2_W, a2_b):
    f32 = jnp.float32
    x_pad = jnp.pad(x, ((0, NPAD - N), (0, 0)))
    def _wp(W, a_s, a_d):
        return jnp.concatenate(
            [W, W @ _headmat(a_s), W @ _headmat(a_d)], axis=1)

    W0p = _wp(gat_W0, att_src0, att_dst0)
    W1p = _wp(gat_W1, att_src1, att_dst1)
    W2p = _wp(gat_W2, att_src2, att_dst2)

    tile64 = pl.BlockSpec((T, 64), lambda i: (i, 0))
    tile72 = pl.BlockSpec((T, 72), lambda i: (i, 0))

    def full(a):
        return pl.BlockSpec(a.shape, lambda i: tuple(0 for _ in a.shape))

    xv_pad, hsd0 = pl.pallas_call(
        _enc_kernel,
        grid=(NT,),
        in_specs=[pl.BlockSpec((T, 8), lambda i: (i, 0)),
                  full(enc_W), full(_row(enc_b)), full(_row(enc_g)),
                  full(_row(enc_beta)), full(_row(virtual_node)), full(W0p)],
        out_specs=[tile64, tile72],
        out_shape=[jax.ShapeDtypeStruct((NPAD, 64), f32),
                   jax.ShapeDtypeStruct((NPAD, 72), f32)],
    )(x_pad, enc_W, _row(enc_b), _row(enc_g), _row(enc_beta),
      _row(virtual_node), W0p)

    idt = edge_index.dtype
    nodes = jnp.arange(N, dtype=idt)
    vfull = jnp.full((N,), N, dtype=idt)
    loop = jnp.arange(NP1, dtype=idt)
    s_full = jnp.concatenate([edge_index[0], vfull, nodes, loop])
    d_full = jnp.concatenate([edge_index[1], nodes, vfull, loop])

    def edge_pass(hsd):
        h = hsd[:NP1, :64]
        sl = hsd[:NP1, 64:68]
        dl = hsd[:NP1, 68:72]
        e = jnp.exp(jax.nn.leaky_relu(sl[s_full] + dl[d_full], 0.2))
        msg = (h.reshape(NP1, 4, 16)[s_full] * e[:, :, None]).reshape(-1, 64)
        payload = jnp.concatenate([msg, e], axis=1)
        seg = jax.ops.segment_sum(payload, d_full, num_segments=NP1)
        return jnp.pad(seg, ((0, NPAD - NP1), (0, 0)))

    def fused(seg, hprev, b, g, bt, Wn):
        return pl.pallas_call(
            _fused_kernel,
            grid=(NT,),
            in_specs=[pl.BlockSpec((T, 68), lambda i: (i, 0)), tile64,
                      full(_row(b)), full(_row(g)),
                      full(_row(bt)), full(Wn)],
            out_specs=[tile64, tile72],
            out_shape=[jax.ShapeDtypeStruct((NPAD, 64), f32),
                       jax.ShapeDtypeStruct((NPAD, 72), f32)],
        )(seg, hprev, _row(b), _row(g), _row(bt), Wn)

    seg0 = edge_pass(hsd0)
    h1_pad, hsd1 = fused(seg0, xv_pad, gat_b0, ln_g0, ln_b0, W1p)
    seg1 = edge_pass(hsd1)
    h2_pad, hsd2 = fused(seg1, h1_pad, gat_b1, ln_g1, ln_b1, W2p)

    rowv = jax.lax.dynamic_slice(hsd2, (N, 0), (1, 72))
    state = pl.pallas_call(
        _flash_kernel,
        grid=(2, NT),
        in_specs=[pl.BlockSpec((T, 72), lambda p, i: (i, 0)),
                  pl.BlockSpec((1, 72), lambda p, i: (0, 0))],
        out_specs=pl.BlockSpec((8, 128), lambda p, i: (0, 0)),
        out_shape=jax.ShapeDtypeStruct((8, 128), f32),
    )(hsd2, rowv)

    jk192 = jnp.concatenate(
        [xv_pad[N:N + 1], h1_pad[N:N + 1], h2_pad[N:N + 1]], axis=1)
    v2Wp = jnp.pad(v2_W, ((0, 0), (0, 127)))
    v2bp = jnp.pad(_row(v2_b), ((0, 0), (0, 127)))
    a2Wp = jnp.pad(a2_W, ((0, 0), (0, 119)))
    a2bp = jnp.pad(_row(a2_b), ((0, 0), (0, 119)))

    out = pl.pallas_call(
        _head_kernel,
        out_shape=jax.ShapeDtypeStruct((8, 128), f32),
    )(state, jk192, h2_pad[N:N + 1], _row(gat_b2), _row(ln_g2), _row(ln_b2),
      agent_features, ag_W, _row(ag_b), _row(ag_g), _row(ag_beta),
      v1_W[:192], v1_W[192:256], v1_W[256:], _row(v1_b), _row(v_g),
      _row(v_beta), v2Wp, v2bp,
      a1_W[:192], a1_W[192:256], a1_W[256:], _row(a1_b), _row(a_g),
      _row(a_beta), a2Wp, a2bp)
    return out[0:1, 0:9]
